# baseline (speedup 1.0000x reference)
import functools

import jax
import jax.numpy as jnp
from jax import lax
from jax.experimental import pallas as pl
from jax.experimental.pallas import tpu as pltpu
from jax.experimental.pallas import tpu_sc as plsc

B = 1024
S = 200
D = 64
N = B * S
NW = 32
PER_W = N // NW
C = 640
NCHUNK = PER_W // C
CB = C // 128
U = 4
EPS = 1e-6
CW = C * D
SEGOFF = S * D


def _rsqrt_vec(x):
    i = lax.bitcast_convert_type(x, jnp.int32)
    i = jnp.full((16,), jnp.int32(0x5F3759DF)) - lax.shift_right_logical(i, 1)
    y = lax.bitcast_convert_type(i, jnp.float32)
    h = x * 0.5
    for _ in range(3):
        y = y * (1.5 - h * y * y)
    return y


def _body(tok_idx, seg_idx, table, pos, segtab, gamma, beta, out,
          tok_a, tok_b, cid_a, cid_b, rows_a, rows_b,
          combo_v, segtab_v, gam_v, bet_v,
          sem_ga, sem_gb, sem_sa, sem_sb):
    wid = lax.axis_index("s") * 2 + lax.axis_index("c")
    wbase = wid * PER_W

    pltpu.sync_copy(pos, combo_v.at[pl.ds(SEGOFF, SEGOFF)])
    pltpu.sync_copy(segtab, segtab_v)
    pltpu.sync_copy(gamma, gam_v)
    pltpu.sync_copy(beta, bet_v)

    s0 = [segtab_v[pl.ds(16 * j, 16)] for j in range(4)]
    s1 = [segtab_v[pl.ds(D + 16 * j, 16)] for j in range(4)]
    gv = [gam_v[pl.ds(16 * j, 16)] for j in range(4)]
    bv = [bet_v[pl.ds(16 * j, 16)] for j in range(4)]

    def combo_body(p, _):
        o1 = pl.multiple_of(SEGOFF + p * D, 64)
        o0 = pl.multiple_of(p * D, 64)
        for j in range(4):
            pv = combo_v[pl.ds(o1 + 16 * j, 16)]
            combo_v[pl.ds(o0 + 16 * j, 16)] = pv + s0[j]
            combo_v[pl.ds(o1 + 16 * j, 16)] = pv + s1[j]
        return ()

    lax.fori_loop(0, S, combo_body, (), unroll=False)

    lane = lax.iota(jnp.int32, 16)
    dnums = lax.GatherDimensionNumbers(
        offset_dims=(), collapsed_slice_dims=(0,), start_index_map=(0,))
    perms = [
        lax.bitwise_xor(lane, jnp.full((16,), jnp.int32(m))).reshape(16, 1)
        for m in (8, 4, 2, 1)
    ]
    lane_j = [lane + 16 * j for j in range(4)]

    def allsum(v):
        for perm in perms:
            v = v + lax.gather(
                v, perm, dnums, (1,),
                mode=lax.GatherScatterMode.PROMISE_IN_BOUNDS)
        return v

    def stage_and_fire(k, tok_v, cid_v, rows_v, sem):
        base = pl.multiple_of(wbase + k * C, 128)
        pltpu.sync_copy(tok_idx.at[pl.ds(base, C)], tok_v)
        pltpu.sync_copy(seg_idx.at[pl.ds(base, C)], cid_v)
        cps = [
            pltpu.async_copy(
                table.at[tok_v.at[pl.ds(j * 128, 128)]],
                rows_v.at[pl.ds(j * 128, 128)],
                sem,
            )
            for j in range(CB)
        ]
        return cps

    def compute(k, cid_v, rows_vf, rows_v2):
        base = wbase + k * C

        def prep_body(i, _):
            off = pl.multiple_of(i * 16, 16)
            segv = cid_v[pl.ds(off, 16)]
            posv = lax.rem(jnp.full((16,), base + i * 16) + lane,
                           jnp.full((16,), S))
            cid_v[pl.ds(off, 16)] = (segv * SEGOFF + posv * D)
            return ()

        lax.fori_loop(0, C // 16, prep_body, (), unroll=False)

        def row_body(i, _):
            ri = jnp.full((16,), i * U, dtype=jnp.int32)
            for u in range(U):
                ru = ri + u
                cid = plsc.load_gather(cid_v, [ru])
                x = []
                for j in range(4):
                    t = plsc.load_gather(rows_vf, [ru, lane_j[j]])
                    cmb = plsc.load_gather(combo_v, [cid + lane_j[j]])
                    x.append(t + cmb)
                ssum = allsum((x[0] + x[1]) + (x[2] + x[3]))
                qsum = allsum(
                    (x[0] * x[0] + x[1] * x[1])
                    + (x[2] * x[2] + x[3] * x[3])
                )
                mb = ssum * (1.0 / D)
                var = qsum * (1.0 / D) - mb * mb
                rb_ = _rsqrt_vec(var + EPS)
                for j in range(4):
                    o = (x[j] - mb) * rb_ * gv[j] + bv[j]
                    plsc.store_scatter(rows_vf, [ru, lane_j[j]], o)
            return ()

        lax.fori_loop(0, C // U, row_body, (), unroll=False)

    def start_store(k, rows_v2, sem):
        base = pl.multiple_of(wbase + k * C, 128)
        return pltpu.make_async_copy(rows_v2, out.at[pl.ds(base, C)], sem)

    def wait_store(k, rows_v2, sem):
        base = pl.multiple_of(wbase + k * C, 128)
        pltpu.make_async_copy(rows_v2, out.at[pl.ds(base, C)], sem).wait()

    rows_a2 = rows_a
    rows_b2 = rows_b
    rows_af = rows_a
    rows_bf = rows_b

    cps = stage_and_fire(0, tok_a, cid_a, rows_a2, sem_ga)
    for cp in cps:
        cp.wait()

    def super_body(t, _):
        k0 = t * 2
        cpsb = stage_and_fire(k0 + 1, tok_b, cid_b, rows_b2, sem_gb)
        compute(k0, cid_a, rows_af, rows_a2)
        sa = start_store(k0, rows_a2, sem_sa)
        sa.start()
        for cp in cpsb:
            cp.wait()
        compute(k0 + 1, cid_b, rows_bf, rows_b2)
        sb = start_store(k0 + 1, rows_b2, sem_sb)
        sb.start()
        wait_store(k0, rows_a2, sem_sa)

        @pl.when(t < NCHUNK // 2 - 1)
        def _():
            cpsa = stage_and_fire(k0 + 2, tok_a, cid_a, rows_a2, sem_ga)
            for cp in cpsa:
                cp.wait()

        wait_store(k0 + 1, rows_b2, sem_sb)
        return ()

    lax.fori_loop(0, NCHUNK // 2, super_body, (), unroll=False)


def kernel(token_input, segment_input, token_table, segment_table, pos_table,
           gamma, beta):
    tok1d = token_input.reshape(N)
    seg1d = segment_input.reshape(N)
    pos1d = pos_table.reshape(S * D)
    seg1t = segment_table.reshape(2 * D)

    mesh = plsc.VectorSubcoreMesh(core_axis_name="c", subcore_axis_name="s")
    run = functools.partial(
        pl.kernel,
        mesh=mesh,
        compiler_params=pltpu.CompilerParams(
            use_tc_tiling_on_sc=False, needs_layout_passes=False),
        out_type=jax.ShapeDtypeStruct((N, D), jnp.float32),
        scratch_types=[
            pltpu.VMEM((C,), jnp.int32),
            pltpu.VMEM((C,), jnp.int32),
            pltpu.VMEM((C,), jnp.int32),
            pltpu.VMEM((C,), jnp.int32),
            pltpu.VMEM((C, D), jnp.float32),
            pltpu.VMEM((C, D), jnp.float32),
            pltpu.VMEM((2 * S * D,), jnp.float32),
            pltpu.VMEM((2 * D,), jnp.float32),
            pltpu.VMEM((D,), jnp.float32),
            pltpu.VMEM((D,), jnp.float32),
            pltpu.SemaphoreType.DMA,
            pltpu.SemaphoreType.DMA,
            pltpu.SemaphoreType.DMA,
            pltpu.SemaphoreType.DMA,
        ],
    )(_body)
    out = run(tok1d, seg1d, token_table, pos1d, seg1t, gamma, beta)
    return out.reshape(B, S, D)
